# baseline (speedup 1.0000x reference)
import functools

import jax
import jax.numpy as jnp
from jax import lax
from jax.experimental import pallas as pl
from jax.experimental.pallas import tpu as pltpu
from jax.experimental.pallas import tpu_sc as plsc

M = 1048576
NUM_SAMPLES = 16
EPS = 1e-08
INV_S = 1.0 / NUM_SAMPLES

NC = 2
NS = 16
NW = NC * NS
CHUNK = M // NW
SUB = 2048
NSUB = CHUNK // SUB
LANES = 16

_LN2 = 0.6931471805599453
_SQRT2 = 1.4142135623730951


def _fast_log(r):
    bits = lax.bitcast_convert_type(r, jnp.int32)
    ef = (lax.shift_right_arithmetic(bits, 23) - 127).astype(jnp.float32)
    m = lax.bitcast_convert_type(
        lax.bitwise_or(lax.bitwise_and(bits, 0x007FFFFF), 0x3F800000),
        jnp.float32)
    cond = m > _SQRT2
    m = jnp.where(cond, m * 0.5, m)
    ef = jnp.where(cond, ef + 1.0, ef)
    t = (m - 1.0) / (m + 1.0)
    t2 = t * t
    ln_m = 2.0 * t * (1.0 + t2 * (1.0 / 3.0 + t2 * (1.0 / 5.0 + t2 * (1.0 / 7.0))))
    return ef * _LN2 + ln_m


def _sc_body(q_hbm, u_hbm, out_hbm, q_buf, u_buf, acc_buf, sem_q0, sem_q1,
             sem_u0, sem_u1, sem_out):
    wid = lax.axis_index("c") * NS + lax.axis_index("s")
    base = wid * CHUNK
    sem_q = (sem_q0, sem_q1)
    sem_u = (sem_u0, sem_u1)

    def start(off, b):
        pltpu.make_async_copy(
            q_hbm.at[pl.ds(off, SUB)], q_buf.at[b], sem_q[b]).start()
        pltpu.make_async_copy(
            u_hbm.at[:, pl.ds(off, SUB)], u_buf.at[b], sem_u[b]).start()

    def wait_for(b):
        pltpu.make_async_copy(
            q_hbm.at[pl.ds(0, SUB)], q_buf.at[b], sem_q[b]).wait()
        pltpu.make_async_copy(
            u_hbm.at[:, pl.ds(0, SUB)], u_buf.at[b], sem_u[b]).wait()

    def compute(b, acc):
        def body(i, acc):
            o = i * LANES
            q16 = q_buf[b, pl.ds(o, LANES)]
            cnt = jnp.zeros((LANES,), jnp.float32)
            for k in range(NUM_SAMPLES):
                u16 = u_buf[b, k, pl.ds(o, LANES)]
                cnt = jnp.where(u16 < q16, cnt + 1.0, cnt)
            p = jnp.clip(q16, EPS, 1.0 - EPS)
            w = _fast_log(p / (1.0 - p))
            return acc + (cnt * INV_S - p) * w
        return lax.fori_loop(0, SUB // LANES, body, acc)

    start(base, 0)

    def outer(g, acc):
        for b in range(2):
            s = 2 * g + b
            wait_for(b)

            @pl.when(s + 1 < NSUB)
            def _prefetch():
                start(base + (s + 1) * SUB, 1 - b)

            acc = compute(b, acc)
        return acc

    acc = lax.fori_loop(0, NSUB // 2, outer, jnp.zeros((LANES,), jnp.float32))
    acc_buf[...] = acc
    out_copy = pltpu.make_async_copy(acc_buf, out_hbm.at[wid], sem_out)
    out_copy.start()
    out_copy.wait()


@jax.jit
def _elbo(q_probs, u):
    mesh = plsc.VectorSubcoreMesh(core_axis_name="c", subcore_axis_name="s",
                                  num_cores=NC, num_subcores=NS)
    partials = pl.kernel(
        _sc_body,
        out_type=jax.ShapeDtypeStruct((NW, LANES), jnp.float32),
        mesh=mesh,
        scratch_types=[
            pltpu.VMEM((2, SUB), jnp.float32),
            pltpu.VMEM((2, NUM_SAMPLES, SUB), jnp.float32),
            pltpu.VMEM((LANES,), jnp.float32),
            pltpu.SemaphoreType.DMA,
            pltpu.SemaphoreType.DMA,
            pltpu.SemaphoreType.DMA,
            pltpu.SemaphoreType.DMA,
            pltpu.SemaphoreType.DMA,
        ],
    )(q_probs, u)
    return jnp.sum(partials)


def kernel(q_probs, u):
    return _elbo(q_probs, u)


# speedup vs baseline: 1.0206x; 1.0206x over previous
import jax
import jax.numpy as jnp
from jax import lax
from jax.experimental import pallas as pl
from jax.experimental.pallas import tpu as pltpu
from jax.experimental.pallas import tpu_sc as plsc

M = 1048576
NUM_SAMPLES = 16
EPS = 1e-08
INV_S = 1.0 / NUM_SAMPLES

NC = 2
NS = 16
NW = NC * NS
CHUNK = M // NW
SUB = 2048
NSUB = CHUNK // SUB
LANES = 16

_LN2 = 0.6931471805599453
_SQRT2 = 1.4142135623730951


def _fast_log(r):
    bits = lax.bitcast_convert_type(r, jnp.int32)
    ef = (lax.shift_right_arithmetic(bits, 23) - 127).astype(jnp.float32)
    m = lax.bitcast_convert_type(
        lax.bitwise_or(lax.bitwise_and(bits, 0x007FFFFF), 0x3F800000),
        jnp.float32)
    cond = m > _SQRT2
    m = jnp.where(cond, m * 0.5, m)
    ef = jnp.where(cond, ef + 1.0, ef)
    t = (m - 1.0) / (m + 1.0)
    t2 = t * t
    ln_m = 2.0 * t * (1.0 + t2 * (1.0 / 3.0 + t2 * (1.0 / 5.0 + t2 * (1.0 / 7.0))))
    return ef * _LN2 + ln_m


def _sc_body(q_hbm, u_hbm, out_hbm, q_buf0, q_buf1, u_buf0, u_buf1, acc_buf,
             sem_q0, sem_q1, sem_u0, sem_u1, sem_out):
    wid = lax.axis_index("c") * NS + lax.axis_index("s")
    base = wid * CHUNK
    q_bufs = (q_buf0, q_buf1)
    u_bufs = (u_buf0, u_buf1)
    sem_q = (sem_q0, sem_q1)
    sem_u = (sem_u0, sem_u1)

    def start(off, b):
        pltpu.make_async_copy(
            q_hbm.at[pl.ds(off, SUB)], q_bufs[b], sem_q[b]).start()
        pltpu.make_async_copy(
            u_hbm.at[:, pl.ds(off, SUB)], u_bufs[b], sem_u[b]).start()

    def wait_for(b):
        pltpu.make_async_copy(
            q_hbm.at[pl.ds(0, SUB)], q_bufs[b], sem_q[b]).wait()
        pltpu.make_async_copy(
            u_hbm.at[:, pl.ds(0, SUB)], u_bufs[b], sem_u[b]).wait()

    def compute(b, acc):
        q_buf = q_bufs[b]
        u_buf = u_bufs[b]

        @plsc.parallel_loop(0, SUB, step=LANES, unroll=4, carry=acc)
        def body(o, acc):
            q16 = q_buf[pl.ds(o, LANES)]
            cnts = []
            for g in range(4):
                c = jnp.zeros((LANES,), jnp.float32)
                for k in range(4 * g, 4 * g + 4):
                    u16 = u_buf[k, pl.ds(o, LANES)]
                    c = jnp.where(u16 < q16, c + 1.0, c)
                cnts.append(c)
            cnt = (cnts[0] + cnts[1]) + (cnts[2] + cnts[3])
            p = jnp.clip(q16, EPS, 1.0 - EPS)
            w = _fast_log(p / (1.0 - p))
            return acc + (cnt * INV_S - p) * w

        return body

    start(base, 0)

    def outer(g, acc):
        for b in range(2):
            s = 2 * g + b
            wait_for(b)

            @pl.when(s + 1 < NSUB)
            def _prefetch():
                start(base + (s + 1) * SUB, 1 - b)

            acc = compute(b, acc)
        return acc

    acc = lax.fori_loop(0, NSUB // 2, outer, jnp.zeros((LANES,), jnp.float32))
    acc_buf[...] = acc
    out_copy = pltpu.make_async_copy(acc_buf, out_hbm.at[wid], sem_out)
    out_copy.start()
    out_copy.wait()


@jax.jit
def _elbo(q_probs, u):
    mesh = plsc.VectorSubcoreMesh(core_axis_name="c", subcore_axis_name="s",
                                  num_cores=NC, num_subcores=NS)
    partials = pl.kernel(
        _sc_body,
        out_type=jax.ShapeDtypeStruct((NW, LANES), jnp.float32),
        mesh=mesh,
        scratch_types=[
            pltpu.VMEM((SUB,), jnp.float32),
            pltpu.VMEM((SUB,), jnp.float32),
            pltpu.VMEM((NUM_SAMPLES, SUB), jnp.float32),
            pltpu.VMEM((NUM_SAMPLES, SUB), jnp.float32),
            pltpu.VMEM((LANES,), jnp.float32),
            pltpu.SemaphoreType.DMA,
            pltpu.SemaphoreType.DMA,
            pltpu.SemaphoreType.DMA,
            pltpu.SemaphoreType.DMA,
            pltpu.SemaphoreType.DMA,
        ],
    )(q_probs, u)
    return jnp.sum(partials)


def kernel(q_probs, u):
    return _elbo(q_probs, u)


# speedup vs baseline: 1.0443x; 1.0232x over previous
import jax
import jax.numpy as jnp
from jax import lax
from jax.experimental import pallas as pl
from jax.experimental.pallas import tpu as pltpu
from jax.experimental.pallas import tpu_sc as plsc

M = 1048576
NUM_SAMPLES = 16
EPS = 1e-08
INV_S = 1.0 / NUM_SAMPLES

NC = 2
NS = 16
NW = NC * NS
CHUNK = M // NW
SUB = 2048
NSUB = CHUNK // SUB
LANES = 16

_LN2 = 0.6931471805599453
_SQRT2 = 1.4142135623730951


def _fast_log(r):
    bits = lax.bitcast_convert_type(r, jnp.int32)
    ef = (lax.shift_right_arithmetic(bits, 23) - 127).astype(jnp.float32)
    m = lax.bitcast_convert_type(
        lax.bitwise_or(lax.bitwise_and(bits, 0x007FFFFF), 0x3F800000),
        jnp.float32)
    cond = m > _SQRT2
    m = jnp.where(cond, m * 0.5, m)
    ef = jnp.where(cond, ef + 1.0, ef)
    t = (m - 1.0) / (m + 1.0)
    t2 = t * t
    ln_m = 2.0 * t * (1.0 + t2 * (1.0 / 3.0 + t2 * (1.0 / 5.0 + t2 * (1.0 / 7.0))))
    return ef * _LN2 + ln_m


def _sc_body(q_hbm, u_hbm, out_hbm, q_buf0, q_buf1, u_buf0, u_buf1, acc_buf,
             sem_q0, sem_q1, sem_u0, sem_u1, sem_out):
    wid = lax.axis_index("c") * NS + lax.axis_index("s")
    base = wid * CHUNK
    q_bufs = (q_buf0, q_buf1)
    u_bufs = (u_buf0, u_buf1)
    sem_q = (sem_q0, sem_q1)
    sem_u = (sem_u0, sem_u1)

    def start(off, b):
        pltpu.make_async_copy(
            q_hbm.at[pl.ds(off, SUB)], q_bufs[b], sem_q[b]).start()
        pltpu.make_async_copy(
            u_hbm.at[:, pl.ds(off, SUB)], u_bufs[b], sem_u[b]).start()

    def wait_for(b):
        pltpu.make_async_copy(
            q_hbm.at[pl.ds(0, SUB)], q_bufs[b], sem_q[b]).wait()
        pltpu.make_async_copy(
            u_hbm.at[:, pl.ds(0, SUB)], u_bufs[b], sem_u[b]).wait()

    def compute(b, acc):
        q_buf = q_bufs[b]
        u_buf = u_bufs[b]

        @plsc.parallel_loop(0, SUB, step=LANES, unroll=8, carry=acc)
        def body(o, acc):
            q16 = q_buf[pl.ds(o, LANES)]
            cnts = []
            for g in range(4):
                c = jnp.zeros((LANES,), jnp.float32)
                for k in range(4 * g, 4 * g + 4):
                    u16 = u_buf[k, pl.ds(o, LANES)]
                    c = jnp.where(u16 < q16, c + 1.0, c)
                cnts.append(c)
            cnt = (cnts[0] + cnts[1]) + (cnts[2] + cnts[3])
            p = jnp.clip(q16, EPS, 1.0 - EPS)
            w = _fast_log(p / (1.0 - p))
            return acc + (cnt * INV_S - p) * w

        return body

    start(base, 0)

    def outer(g, acc):
        for b in range(2):
            s = 2 * g + b
            wait_for(b)

            @pl.when(s + 1 < NSUB)
            def _prefetch():
                start(base + (s + 1) * SUB, 1 - b)

            acc = compute(b, acc)
        return acc

    acc = lax.fori_loop(0, NSUB // 2, outer, jnp.zeros((LANES,), jnp.float32))
    acc_buf[...] = acc
    out_copy = pltpu.make_async_copy(acc_buf, out_hbm.at[wid], sem_out)
    out_copy.start()
    out_copy.wait()


@jax.jit
def _elbo(q_probs, u):
    mesh = plsc.VectorSubcoreMesh(core_axis_name="c", subcore_axis_name="s",
                                  num_cores=NC, num_subcores=NS)
    partials = pl.kernel(
        _sc_body,
        out_type=jax.ShapeDtypeStruct((NW, LANES), jnp.float32),
        mesh=mesh,
        scratch_types=[
            pltpu.VMEM((SUB,), jnp.float32),
            pltpu.VMEM((SUB,), jnp.float32),
            pltpu.VMEM((NUM_SAMPLES, SUB), jnp.float32),
            pltpu.VMEM((NUM_SAMPLES, SUB), jnp.float32),
            pltpu.VMEM((LANES,), jnp.float32),
            pltpu.SemaphoreType.DMA,
            pltpu.SemaphoreType.DMA,
            pltpu.SemaphoreType.DMA,
            pltpu.SemaphoreType.DMA,
            pltpu.SemaphoreType.DMA,
        ],
    )(q_probs, u)
    return jnp.sum(partials)


def kernel(q_probs, u):
    return _elbo(q_probs, u)


# speedup vs baseline: 1.1681x; 1.1185x over previous
import jax
import jax.numpy as jnp
from jax import lax
from jax.experimental import pallas as pl
from jax.experimental.pallas import tpu as pltpu
from jax.experimental.pallas import tpu_sc as plsc

M = 1048576
NUM_SAMPLES = 16
EPS = 1e-08
INV_S = 1.0 / NUM_SAMPLES

TC_ROWS = 8
SC_ROWS = NUM_SAMPLES - TC_ROWS

NC = 2
NS = 16
NW = NC * NS
CHUNK = M // NW
SUB = 4096
NSUB = CHUNK // SUB
LANES = 16

_LN2 = 0.6931471805599453
_SQRT2 = 1.4142135623730951

BLK = 65536
GRID = M // BLK


def _fast_log(r):
    bits = lax.bitcast_convert_type(r, jnp.int32)
    ef = (lax.shift_right_arithmetic(bits, 23) - 127).astype(jnp.float32)
    m = lax.bitcast_convert_type(
        lax.bitwise_or(lax.bitwise_and(bits, 0x007FFFFF), 0x3F800000),
        jnp.float32)
    cond = m > _SQRT2
    m = jnp.where(cond, m * 0.5, m)
    ef = jnp.where(cond, ef + 1.0, ef)
    t = (m - 1.0) / (m + 1.0)
    t2 = t * t
    ln_m = 2.0 * t * (1.0 + t2 * (1.0 / 3.0 + t2 * (1.0 / 5.0 + t2 * (1.0 / 7.0))))
    return ef * _LN2 + ln_m


def _sc_body(q_hbm, u_hbm, out_hbm, q_buf0, q_buf1, u_buf0, u_buf1, acc_buf,
             sem_q0, sem_q1, sem_u0, sem_u1, sem_out):
    wid = lax.axis_index("c") * NS + lax.axis_index("s")
    base = wid * CHUNK
    q_bufs = (q_buf0, q_buf1)
    u_bufs = (u_buf0, u_buf1)
    sem_q = (sem_q0, sem_q1)
    sem_u = (sem_u0, sem_u1)

    def start(off, b):
        pltpu.make_async_copy(
            q_hbm.at[pl.ds(off, SUB)], q_bufs[b], sem_q[b]).start()
        pltpu.make_async_copy(
            u_hbm.at[pl.ds(TC_ROWS, SC_ROWS), pl.ds(off, SUB)],
            u_bufs[b], sem_u[b]).start()

    def wait_for(b):
        pltpu.make_async_copy(
            q_hbm.at[pl.ds(0, SUB)], q_bufs[b], sem_q[b]).wait()
        pltpu.make_async_copy(
            u_hbm.at[pl.ds(TC_ROWS, SC_ROWS), pl.ds(0, SUB)],
            u_bufs[b], sem_u[b]).wait()

    def compute(b, acc):
        q_buf = q_bufs[b]
        u_buf = u_bufs[b]

        @plsc.parallel_loop(0, SUB, step=LANES, unroll=4, carry=acc)
        def body(o, acc):
            q16 = q_buf[pl.ds(o, LANES)]
            cnts = []
            for g in range(SC_ROWS // 4):
                c = jnp.zeros((LANES,), jnp.float32)
                for k in range(4 * g, 4 * g + 4):
                    u16 = u_buf[k, pl.ds(o, LANES)]
                    c = jnp.where(u16 < q16, c + 1.0, c)
                cnts.append(c)
            cnt = cnts[0] + cnts[1]
            p = jnp.clip(q16, EPS, 1.0 - EPS)
            w = _fast_log(p / (1.0 - p))
            return acc + (cnt * INV_S) * w

        return body

    start(base, 0)

    def outer(g, acc):
        for b in range(2):
            s = 2 * g + b
            wait_for(b)

            @pl.when(s + 1 < NSUB)
            def _prefetch():
                start(base + (s + 1) * SUB, 1 - b)

            acc = compute(b, acc)
        return acc

    acc = lax.fori_loop(0, NSUB // 2, outer, jnp.zeros((LANES,), jnp.float32))
    acc_buf[...] = acc
    out_copy = pltpu.make_async_copy(acc_buf, out_hbm.at[wid], sem_out)
    out_copy.start()
    out_copy.wait()


def _tc_block(q_ref, u_ref, out_ref):
    i = pl.program_id(0)
    q = q_ref[0]
    p = jnp.clip(q, EPS, 1.0 - EPS)
    w = jnp.log(p) - jnp.log(1.0 - p)
    u = u_ref[...]
    s_cnt = jnp.sum(jnp.where(u < q, w, 0.0))
    s = s_cnt * INV_S - jnp.sum(p * w)

    @pl.when(i == 0)
    def _init():
        out_ref[...] = jnp.zeros((1, 1), jnp.float32)

    out_ref[...] += s


@jax.jit
def _elbo(q_probs, u):
    mesh = plsc.VectorSubcoreMesh(core_axis_name="c", subcore_axis_name="s",
                                  num_cores=NC, num_subcores=NS)
    sc_partials = pl.kernel(
        _sc_body,
        out_type=jax.ShapeDtypeStruct((NW, LANES), jnp.float32),
        mesh=mesh,
        scratch_types=[
            pltpu.VMEM((SUB,), jnp.float32),
            pltpu.VMEM((SUB,), jnp.float32),
            pltpu.VMEM((SC_ROWS, SUB), jnp.float32),
            pltpu.VMEM((SC_ROWS, SUB), jnp.float32),
            pltpu.VMEM((LANES,), jnp.float32),
            pltpu.SemaphoreType.DMA,
            pltpu.SemaphoreType.DMA,
            pltpu.SemaphoreType.DMA,
            pltpu.SemaphoreType.DMA,
            pltpu.SemaphoreType.DMA,
        ],
    )(q_probs, u)

    q2 = q_probs.reshape(GRID, 1, BLK)
    tc_out = pl.pallas_call(
        _tc_block,
        grid=(GRID,),
        in_specs=[
            pl.BlockSpec((1, 1, BLK), lambda i: (i, 0, 0)),
            pl.BlockSpec((TC_ROWS, BLK), lambda i: (0, i)),
        ],
        out_specs=pl.BlockSpec((1, 1), lambda i: (0, 0)),
        out_shape=jax.ShapeDtypeStruct((1, 1), jnp.float32),
    )(q2, u)

    return tc_out[0, 0] + jnp.sum(sc_partials)


def kernel(q_probs, u):
    return _elbo(q_probs, u)


# speedup vs baseline: 1.2051x; 1.0317x over previous
import jax
import jax.numpy as jnp
from jax import lax
from jax.experimental import pallas as pl
from jax.experimental.pallas import tpu as pltpu
from jax.experimental.pallas import tpu_sc as plsc

M = 1048576
NUM_SAMPLES = 16
EPS = 1e-08
INV_S = 1.0 / NUM_SAMPLES

TC_ROWS = 8
SC_ROWS = NUM_SAMPLES - TC_ROWS

NC = 2
NS = 16
NW = NC * NS
CHUNK = M // NW
SUB = 4096
NSUB = CHUNK // SUB
LANES = 16

_LN2 = 0.6931471805599453
_SQRT2 = 1.4142135623730951

BLK = 65536
GRID = M // BLK


def _fast_log(r):
    bits = lax.bitcast_convert_type(r, jnp.int32)
    ef = (lax.shift_right_arithmetic(bits, 23) - 127).astype(jnp.float32)
    m = lax.bitcast_convert_type(
        lax.bitwise_or(lax.bitwise_and(bits, 0x007FFFFF), 0x3F800000),
        jnp.float32)
    cond = m > _SQRT2
    m = jnp.where(cond, m * 0.5, m)
    ef = jnp.where(cond, ef + 1.0, ef)
    t = (m - 1.0) / (m + 1.0)
    t2 = t * t
    ln_m = 2.0 * t * (1.0 + t2 * (1.0 / 3.0 + t2 * (1.0 / 5.0 + t2 * (1.0 / 7.0))))
    return ef * _LN2 + ln_m


def _sc_body(q_hbm, u_hbm, out_hbm, q_buf0, q_buf1, u_buf0, u_buf1, acc_buf,
             sem_q0, sem_q1, sem_u0, sem_u1, sem_out):
    wid = lax.axis_index("c") * NS + lax.axis_index("s")
    base = wid * CHUNK
    q_bufs = (q_buf0, q_buf1)
    u_bufs = (u_buf0, u_buf1)
    sem_q = (sem_q0, sem_q1)
    sem_u = (sem_u0, sem_u1)

    def start(off, b):
        pltpu.make_async_copy(
            q_hbm.at[pl.ds(off, SUB)], q_bufs[b], sem_q[b]).start()
        pltpu.make_async_copy(
            u_hbm.at[pl.ds(TC_ROWS, SC_ROWS), pl.ds(off, SUB)],
            u_bufs[b], sem_u[b]).start()

    def wait_for(b):
        pltpu.make_async_copy(
            q_hbm.at[pl.ds(0, SUB)], q_bufs[b], sem_q[b]).wait()
        pltpu.make_async_copy(
            u_hbm.at[pl.ds(TC_ROWS, SC_ROWS), pl.ds(0, SUB)],
            u_bufs[b], sem_u[b]).wait()

    def compute(b, acc):
        q_buf = q_bufs[b]
        u_buf = u_bufs[b]

        @plsc.parallel_loop(0, SUB, step=LANES, unroll=2, carry=acc)
        def body(o, acc):
            q16 = q_buf[pl.ds(o, LANES)]
            cnts = []
            for g in range(SC_ROWS // 4):
                c = jnp.zeros((LANES,), jnp.float32)
                for k in range(4 * g, 4 * g + 4):
                    u16 = u_buf[k, pl.ds(o, LANES)]
                    c = jnp.where(u16 < q16, c + 1.0, c)
                cnts.append(c)
            cnt = cnts[0] + cnts[1]
            p = jnp.clip(q16, EPS, 1.0 - EPS)
            w = _fast_log(p / (1.0 - p))
            return acc + (cnt * INV_S) * w

        return body

    start(base, 0)

    def outer(g, acc):
        for b in range(2):
            s = 2 * g + b
            wait_for(b)

            @pl.when(s + 1 < NSUB)
            def _prefetch():
                start(base + (s + 1) * SUB, 1 - b)

            acc = compute(b, acc)
        return acc

    acc = lax.fori_loop(0, NSUB // 2, outer, jnp.zeros((LANES,), jnp.float32))
    acc_buf[...] = acc
    out_copy = pltpu.make_async_copy(acc_buf, out_hbm.at[wid], sem_out)
    out_copy.start()
    out_copy.wait()


def _tc_block(q_ref, u_ref, out_ref):
    i = pl.program_id(0)
    q = q_ref[0]
    p = jnp.clip(q, EPS, 1.0 - EPS)
    w = jnp.log(p) - jnp.log(1.0 - p)
    u = u_ref[...]
    s_cnt = jnp.sum(jnp.where(u < q, w, 0.0))
    s = s_cnt * INV_S - jnp.sum(p * w)

    @pl.when(i == 0)
    def _init():
        out_ref[...] = jnp.zeros((1, 1), jnp.float32)

    out_ref[...] += s


@jax.jit
def _elbo(q_probs, u):
    mesh = plsc.VectorSubcoreMesh(core_axis_name="c", subcore_axis_name="s",
                                  num_cores=NC, num_subcores=NS)
    sc_partials = pl.kernel(
        _sc_body,
        out_type=jax.ShapeDtypeStruct((NW, LANES), jnp.float32),
        mesh=mesh,
        scratch_types=[
            pltpu.VMEM((SUB,), jnp.float32),
            pltpu.VMEM((SUB,), jnp.float32),
            pltpu.VMEM((SC_ROWS, SUB), jnp.float32),
            pltpu.VMEM((SC_ROWS, SUB), jnp.float32),
            pltpu.VMEM((LANES,), jnp.float32),
            pltpu.SemaphoreType.DMA,
            pltpu.SemaphoreType.DMA,
            pltpu.SemaphoreType.DMA,
            pltpu.SemaphoreType.DMA,
            pltpu.SemaphoreType.DMA,
        ],
    )(q_probs, u)

    q2 = q_probs.reshape(GRID, 1, BLK)
    tc_out = pl.pallas_call(
        _tc_block,
        grid=(GRID,),
        in_specs=[
            pl.BlockSpec((1, 1, BLK), lambda i: (i, 0, 0)),
            pl.BlockSpec((TC_ROWS, BLK), lambda i: (0, i)),
        ],
        out_specs=pl.BlockSpec((1, 1), lambda i: (0, 0)),
        out_shape=jax.ShapeDtypeStruct((1, 1), jnp.float32),
    )(q2, u)

    return tc_out[0, 0] + jnp.sum(sc_partials)


def kernel(q_probs, u):
    return _elbo(q_probs, u)


# speedup vs baseline: 2.1280x; 1.7658x over previous
import functools

import jax
import jax.numpy as jnp
from jax.experimental import pallas as pl

M = 1048576
NUM_SAMPLES = 16
EPS = 1e-08
BLK = 131072
GRID = M // BLK
INV_S = 1.0 / NUM_SAMPLES


def _elbo_block(q_ref, u_ref, out_ref):
    i = pl.program_id(0)
    q = q_ref[0]
    p = jnp.clip(q, EPS, 1.0 - EPS)
    w = jnp.log(p) - jnp.log(1.0 - p)
    u = u_ref[...]
    s_cnt = jnp.sum(jnp.where(u < q, w, 0.0))
    s = s_cnt * INV_S - jnp.sum(p * w)

    @pl.when(i == 0)
    def _init():
        out_ref[...] = jnp.zeros((1, 1), jnp.float32)

    out_ref[...] += s


@functools.partial(jax.jit)
def _elbo(q_probs, u):
    q2 = q_probs.reshape(GRID, 1, BLK)
    out = pl.pallas_call(
        _elbo_block,
        grid=(GRID,),
        in_specs=[
            pl.BlockSpec((1, 1, BLK), lambda i: (i, 0, 0)),
            pl.BlockSpec((NUM_SAMPLES, BLK), lambda i: (0, i)),
        ],
        out_specs=pl.BlockSpec((1, 1), lambda i: (0, 0)),
        out_shape=jax.ShapeDtypeStruct((1, 1), jnp.float32),
    )(q2, u)
    return out[0, 0]


def kernel(q_probs, u):
    return _elbo(q_probs, u)


# speedup vs baseline: 2.1656x; 1.0177x over previous
import functools

import jax
import jax.numpy as jnp
from jax.experimental import pallas as pl

M = 1048576
NUM_SAMPLES = 16
EPS = 1e-08
BLK = 262144
GRID = M // BLK
INV_S = 1.0 / NUM_SAMPLES


def _elbo_block(q_ref, u_ref, out_ref):
    i = pl.program_id(0)
    q = q_ref[0]
    p = jnp.clip(q, EPS, 1.0 - EPS)
    w = jnp.log(p) - jnp.log(1.0 - p)
    u = u_ref[...]
    s_cnt = jnp.sum(jnp.where(u < q, w, 0.0))
    s = s_cnt * INV_S - jnp.sum(p * w)

    @pl.when(i == 0)
    def _init():
        out_ref[...] = jnp.zeros((1, 1), jnp.float32)

    out_ref[...] += s


@functools.partial(jax.jit)
def _elbo(q_probs, u):
    q2 = q_probs.reshape(GRID, 1, BLK)
    out = pl.pallas_call(
        _elbo_block,
        grid=(GRID,),
        in_specs=[
            pl.BlockSpec((1, 1, BLK), lambda i: (i, 0, 0)),
            pl.BlockSpec((NUM_SAMPLES, BLK), lambda i: (0, i)),
        ],
        out_specs=pl.BlockSpec((1, 1), lambda i: (0, 0)),
        out_shape=jax.ShapeDtypeStruct((1, 1), jnp.float32),
    )(q2, u)
    return out[0, 0]


def kernel(q_probs, u):
    return _elbo(q_probs, u)
